# baseline (speedup 1.0000x reference)
import functools

import jax
import jax.numpy as jnp
from jax import lax
from jax.experimental import pallas as pl
from jax.experimental.pallas import tpu as pltpu
from jax.experimental.pallas import tpu_sc as plsc

_K = 2
_BM = 256


def _gelu_exact(v):
    return 0.5 * v * (1.0 + jax.lax.erf(v * (2.0 ** -0.5)))


def _sc_gather_rows(x2, row_tok, npc):
    S, D = x2.shape
    info = plsc.get_sparse_core_info()
    NW = info.num_cores * info.num_subcores
    per_w = npc // NW
    CH = 16
    n_ch = per_w // CH
    mesh = plsc.VectorSubcoreMesh(core_axis_name="c", subcore_axis_name="s")

    @functools.partial(
        pl.kernel, mesh=mesh,
        out_type=jax.ShapeDtypeStruct((npc, D), jnp.float32),
        scratch_types=[
            pltpu.VMEM((CH,), jnp.int32),
            pltpu.VMEM((CH, D), jnp.float32),
            pltpu.SemaphoreType.DMA,
        ],
    )
    def k(x_hbm, idx_hbm, out_hbm, idx_v, rows_v, sem):
        wid = lax.axis_index("s") * info.num_cores + lax.axis_index("c")
        base = wid * per_w

        def chunk(c, _):
            off = base + c * CH
            pltpu.sync_copy(idx_hbm.at[pl.ds(off, CH)], idx_v)
            pltpu.async_copy(x_hbm.at[idx_v], rows_v, sem).wait()
            pltpu.sync_copy(rows_v, out_hbm.at[pl.ds(off, CH)])
            return 0

        lax.fori_loop(0, n_ch, chunk, 0)

    return k(x2, row_tok)


def _sc_combine(yw, pos0, pos1, base):
    S, D = base.shape
    info = plsc.get_sparse_core_info()
    NW = info.num_cores * info.num_subcores
    per_w = S // NW
    CH = 16
    n_ch = per_w // CH
    nvec = D // 16
    mesh = plsc.VectorSubcoreMesh(core_axis_name="c", subcore_axis_name="s")

    @functools.partial(
        pl.kernel, mesh=mesh,
        out_type=jax.ShapeDtypeStruct((S, D), jnp.float32),
        scratch_types=[
            pltpu.VMEM((CH,), jnp.int32),
            pltpu.VMEM((CH,), jnp.int32),
            pltpu.VMEM((CH, D), jnp.float32),
            pltpu.VMEM((CH, D), jnp.float32),
            pltpu.VMEM((CH, D), jnp.float32),
            pltpu.SemaphoreType.DMA,
            pltpu.SemaphoreType.DMA,
        ],
    )
    def k(yw_hbm, p0_hbm, p1_hbm, base_hbm, out_hbm,
          i0_v, i1_v, a_v, b_v, s_v, sem0, sem1):
        wid = lax.axis_index("s") * info.num_cores + lax.axis_index("c")
        wbase = wid * per_w

        def chunk(c, _):
            off = wbase + c * CH
            pltpu.sync_copy(p0_hbm.at[pl.ds(off, CH)], i0_v)
            pltpu.sync_copy(p1_hbm.at[pl.ds(off, CH)], i1_v)
            cp_a = pltpu.async_copy(yw_hbm.at[i0_v], a_v, sem0)
            cp_b = pltpu.async_copy(yw_hbm.at[i1_v], b_v, sem1)
            pltpu.sync_copy(base_hbm.at[pl.ds(off, CH)], s_v)
            cp_a.wait()
            cp_b.wait()

            def vec(i, _):
                r = i // nvec
                sl = pl.ds((i % nvec) * 16, 16)
                s_v[r, sl] = s_v[r, sl] + a_v[r, sl] + b_v[r, sl]
                return 0

            lax.fori_loop(0, CH * nvec, vec, 0)
            pltpu.sync_copy(s_v, out_hbm.at[pl.ds(off, CH)])
            return 0

        lax.fori_loop(0, n_ch, chunk, 0)

    return k(yw, pos0, pos1, base)


def _grouped_ffn_body(te_ref, x_ref, w_ref, W1_ref, W2_ref, b1_ref, out_ref):
    x = x_ref[...]
    h = jax.lax.dot_general(x, W1_ref[0], (((1,), (1,)), ((), ())),
                            preferred_element_type=jnp.float32)
    h = _gelu_exact(h + b1_ref[0])
    y = jax.lax.dot_general(h, W2_ref[0], (((1,), (1,)), ((), ())),
                            preferred_element_type=jnp.float32)
    out_ref[...] = w_ref[...] * y


def _shared_body(x_ref, Ws1_ref, Ws2_ref, bs1_ref, out_ref):
    x = x_ref[...]
    h = jax.lax.dot_general(x, Ws1_ref[...], (((1,), (1,)), ((), ())),
                            preferred_element_type=jnp.float32)
    h = _gelu_exact(h + bs1_ref[...])
    y = jax.lax.dot_general(h, Ws2_ref[...], (((1,), (1,)), ((), ())),
                            preferred_element_type=jnp.float32)
    out_ref[...] = y


def kernel(x, gate_w, W1, b1, W2, b2, Ws1, bs1, Ws2, bs2, route_scale):
    original_shape = x.shape
    if x.ndim == 2:
        x = x[:, None, :]
    Bx, Sx, D = x.shape
    E, F, _ = W1.shape

    gate_scores = x @ gate_w.T
    scores = jax.nn.sigmoid(gate_scores) * route_scale
    topk_scores, topk_idx = jax.lax.top_k(scores, _K)
    topk_w = topk_scores / jnp.sum(topk_scores, axis=-1, keepdims=True)

    onehot_f = jax.nn.one_hot(topk_idx, E, dtype=jnp.float32)
    present = jnp.any(onehot_f > 0, axis=(0, 1))
    counts = jnp.sum(present.astype(jnp.float32), axis=0)
    expert_usage = counts / jnp.sum(counts)

    S = Bx * Sx
    P = S * _K
    x2 = x.reshape(S, D)

    eid = topk_idx.reshape(P)
    wv = topk_w.reshape(P)
    tok = (jnp.arange(P, dtype=jnp.int32) // _K)
    oh = jax.nn.one_hot(eid, E, dtype=jnp.int32)
    rank = jnp.cumsum(oh, axis=0) - oh
    rank_within = jnp.take_along_axis(rank, eid[:, None], axis=1)[:, 0]
    ge = jnp.sum(oh, axis=0)
    pe = ((ge + _BM - 1) // _BM) * _BM
    start_pad = jnp.concatenate(
        [jnp.zeros((1,), jnp.int32), jnp.cumsum(pe)]).astype(jnp.int32)
    dest = (start_pad[eid] + rank_within).astype(jnp.int32)

    NPC = P + E * _BM
    row_tok = jnp.zeros((NPC,), jnp.int32).at[dest].set(tok)
    row_w = jnp.zeros((NPC,), jnp.float32).at[dest].set(wv)
    pos = dest.reshape(S, _K)
    pos0 = pos[:, 0]
    pos1 = pos[:, 1]

    blk_start = jnp.arange(NPC // _BM, dtype=jnp.int32) * _BM
    te = jnp.clip(jnp.searchsorted(start_pad[1:], blk_start, side="right"),
                  0, E - 1).astype(jnp.int32)

    xs = _sc_gather_rows(x2, row_tok, NPC)

    G = NPC // _BM
    yw = pl.pallas_call(
        _grouped_ffn_body,
        grid_spec=pltpu.PrefetchScalarGridSpec(
            num_scalar_prefetch=1,
            grid=(G,),
            in_specs=[
                pl.BlockSpec((_BM, D), lambda g, te_r: (g, 0)),
                pl.BlockSpec((_BM, 1), lambda g, te_r: (g, 0)),
                pl.BlockSpec((1, F, D), lambda g, te_r: (te_r[g], 0, 0)),
                pl.BlockSpec((1, D, F), lambda g, te_r: (te_r[g], 0, 0)),
                pl.BlockSpec((1, 1, F), lambda g, te_r: (te_r[g], 0, 0)),
            ],
            out_specs=pl.BlockSpec((_BM, D), lambda g, te_r: (g, 0)),
        ),
        out_shape=jax.ShapeDtypeStruct((NPC, D), jnp.float32),
        compiler_params=pltpu.CompilerParams(
            dimension_semantics=("arbitrary",),
        ),
    )(te, xs, row_w[:, None], W1, W2, b1[:, None, :])

    Bt = min(512, S)
    shared = pl.pallas_call(
        _shared_body,
        grid=(S // Bt,),
        in_specs=[
            pl.BlockSpec((Bt, D), lambda t: (t, 0)),
            pl.BlockSpec((F, D), lambda t: (0, 0)),
            pl.BlockSpec((D, F), lambda t: (0, 0)),
            pl.BlockSpec((1, F), lambda t: (0, 0)),
        ],
        out_specs=pl.BlockSpec((Bt, D), lambda t: (t, 0)),
        out_shape=jax.ShapeDtypeStruct((S, D), jnp.float32),
        compiler_params=pltpu.CompilerParams(
            dimension_semantics=("parallel",),
        ),
    )(x2, Ws1, Ws2, bs1[None, :])

    out = _sc_combine(yw, pos0, pos1, shared)

    w_full = jnp.einsum("bske,bsk->bse", onehot_f, topk_w)
    out = out + w_full.reshape(S, E) @ b2 + bs2[None, :]
    output = out.reshape(original_shape)
    return output, expert_usage, topk_idx
